# baseline (speedup 1.0000x reference)
import functools

import jax
import jax.numpy as jnp
from jax import lax
from jax.experimental import pallas as pl
from jax.experimental.pallas import tpu as pltpu
from jax.experimental.pallas import tpu_sc as plsc

VOCAB = 100000
HIDDEN = 2048
MEM = 128
B, S = 4, 4096
N = B * S


_info = plsc.get_sparse_core_info()
_NC, _NS = _info.num_cores, _info.num_subcores
_NW = _NC * _NS
_NPW = N // _NW
_CHUNK = 128
_NCHUNK = _NPW // _CHUNK


@functools.partial(
    pl.kernel,
    mesh=plsc.VectorSubcoreMesh(core_axis_name="c", subcore_axis_name="s"),
    out_type=jax.ShapeDtypeStruct((N, MEM), jnp.float32),
    scratch_types=[
        pltpu.VMEM((_NPW,), jnp.int32),
        pltpu.VMEM((_NPW, MEM), jnp.float32),
        pltpu.SemaphoreType.DMA,
    ],
)
def _sc_gather(table_hbm, idx_hbm, out_hbm, idx_v, rows_v, sem):
    wid = lax.axis_index("s") * _NC + lax.axis_index("c")
    base = wid * _NPW
    pltpu.sync_copy(idx_hbm.at[pl.ds(base, _NPW)], idx_v)
    for j in range(_NCHUNK):
        pltpu.async_copy(
            table_hbm.at[idx_v.at[pl.ds(j * _CHUNK, _CHUNK)]],
            rows_v.at[pl.ds(j * _CHUNK, _CHUNK)],
            sem,
        ).wait()
    pltpu.sync_copy(rows_v, out_hbm.at[pl.ds(base, _NPW)])


_TB = 512


def _tc_body(hs_ref, e_ref, wg_ref, wo_ref, nw_ref, out_ref):
    hs = hs_ref[...].astype(jnp.bfloat16)
    g = jax.nn.sigmoid(
        lax.dot_general(hs, wg_ref[...].astype(jnp.bfloat16),
                        (((1,), (1,)), ((), ())),
                        preferred_element_type=jnp.float32))
    v = (e_ref[...] + g).astype(jnp.bfloat16)
    y = lax.dot_general(v, wo_ref[...].astype(jnp.bfloat16),
                        (((1,), (1,)), ((), ())),
                        preferred_element_type=jnp.float32)
    var = jnp.mean(y * y, axis=-1, keepdims=True)
    out_ref[...] = y * lax.rsqrt(var + 1e-6) * nw_ref[...]


def kernel(hidden_states, input_ids, memory, W_gate, W_out, norm_w):
    hs = hidden_states.reshape(N, HIDDEN)
    ids = input_ids.astype(jnp.int32).reshape(N)

    e = _sc_gather(memory, ids)

    out = pl.pallas_call(
        _tc_body,
        grid=(N // _TB,),
        in_specs=[
            pl.BlockSpec((_TB, HIDDEN), lambda i: (i, 0)),
            pl.BlockSpec((_TB, MEM), lambda i: (i, 0)),
            pl.BlockSpec((MEM, HIDDEN), lambda i: (0, 0)),
            pl.BlockSpec((HIDDEN, MEM), lambda i: (0, 0)),
            pl.BlockSpec((1, HIDDEN), lambda i: (0, 0)),
        ],
        out_specs=pl.BlockSpec((_TB, HIDDEN), lambda i: (i, 0)),
        out_shape=jax.ShapeDtypeStruct((N, HIDDEN), jnp.float32),
    )(hs, e, W_gate, W_out, norm_w.reshape(1, HIDDEN))

    return out.reshape(B, S, HIDDEN)


# speedup vs baseline: 1.0825x; 1.0825x over previous
import functools

import jax
import jax.numpy as jnp
from jax import lax
from jax.experimental import pallas as pl
from jax.experimental.pallas import tpu as pltpu
from jax.experimental.pallas import tpu_sc as plsc

VOCAB = 100000
HIDDEN = 2048
MEM = 128
B, S = 4, 4096
N = B * S


_info = plsc.get_sparse_core_info()
_NC, _NS = _info.num_cores, _info.num_subcores
_NW = _NC * _NS
_NPW = N // _NW
_CHUNK = 128
_NCHUNK = _NPW // _CHUNK


@functools.partial(
    pl.kernel,
    mesh=plsc.VectorSubcoreMesh(core_axis_name="c", subcore_axis_name="s"),
    out_type=jax.ShapeDtypeStruct((N, MEM), jnp.float32),
    scratch_types=[
        pltpu.VMEM((_NPW,), jnp.int32),
        pltpu.VMEM((_NPW, MEM), jnp.float32),
        pltpu.SemaphoreType.DMA,
    ],
)
def _sc_gather(table_hbm, idx_hbm, out_hbm, idx_v, rows_v, sem):
    wid = lax.axis_index("s") * _NC + lax.axis_index("c")
    base = wid * _NPW
    pltpu.sync_copy(idx_hbm.at[pl.ds(base, _NPW)], idx_v)
    for j in range(_NCHUNK):
        pltpu.async_copy(
            table_hbm.at[idx_v.at[pl.ds(j * _CHUNK, _CHUNK)]],
            rows_v.at[pl.ds(j * _CHUNK, _CHUNK)],
            sem,
        ).wait()
    pltpu.sync_copy(rows_v, out_hbm.at[pl.ds(base, _NPW)])


_TB = 1024


def _tc_body(hs_ref, e_ref, wg_ref, wo_ref, nw_ref, out_ref):
    hs = hs_ref[...].astype(jnp.bfloat16)
    g = jax.nn.sigmoid(
        lax.dot_general(hs, wg_ref[...].astype(jnp.bfloat16),
                        (((1,), (1,)), ((), ())),
                        preferred_element_type=jnp.float32))
    v = (e_ref[...] + g).astype(jnp.bfloat16)
    y = lax.dot_general(v, wo_ref[...].astype(jnp.bfloat16),
                        (((1,), (1,)), ((), ())),
                        preferred_element_type=jnp.float32)
    var = jnp.mean(y * y, axis=-1, keepdims=True)
    out_ref[...] = y * lax.rsqrt(var + 1e-6) * nw_ref[...]


def kernel(hidden_states, input_ids, memory, W_gate, W_out, norm_w):
    hs = hidden_states.reshape(N, HIDDEN)
    ids = input_ids.astype(jnp.int32).reshape(N)

    e = _sc_gather(memory, ids)

    out = pl.pallas_call(
        _tc_body,
        grid=(N // _TB,),
        in_specs=[
            pl.BlockSpec((_TB, HIDDEN), lambda i: (i, 0)),
            pl.BlockSpec((_TB, MEM), lambda i: (i, 0)),
            pl.BlockSpec((MEM, HIDDEN), lambda i: (0, 0)),
            pl.BlockSpec((HIDDEN, MEM), lambda i: (0, 0)),
            pl.BlockSpec((1, HIDDEN), lambda i: (0, 0)),
        ],
        out_specs=pl.BlockSpec((_TB, HIDDEN), lambda i: (i, 0)),
        out_shape=jax.ShapeDtypeStruct((N, HIDDEN), jnp.float32),
    )(hs, e, W_gate, W_out, norm_w.reshape(1, HIDDEN))

    return out.reshape(B, S, HIDDEN)


# speedup vs baseline: 1.1623x; 1.0737x over previous
import functools

import jax
import jax.numpy as jnp
from jax import lax
from jax.experimental import pallas as pl
from jax.experimental.pallas import tpu as pltpu
from jax.experimental.pallas import tpu_sc as plsc

VOCAB = 100000
HIDDEN = 2048
MEM = 128
B, S = 4, 4096
N = B * S


_info = plsc.get_sparse_core_info()
_NC, _NS = _info.num_cores, _info.num_subcores
_NW = _NC * _NS
_NPW = N // _NW
_CHUNK = 128
_NCHUNK = _NPW // _CHUNK


@functools.partial(
    pl.kernel,
    mesh=plsc.VectorSubcoreMesh(core_axis_name="c", subcore_axis_name="s"),
    out_type=jax.ShapeDtypeStruct((N, MEM), jnp.float32),
    scratch_types=[
        pltpu.VMEM((_NPW,), jnp.int32),
        pltpu.VMEM((_NPW, MEM), jnp.float32),
        pltpu.SemaphoreType.DMA,
    ],
)
def _sc_gather(table_hbm, idx_hbm, out_hbm, idx_v, rows_v, sem):
    wid = lax.axis_index("s") * _NC + lax.axis_index("c")
    base = wid * _NPW
    pltpu.sync_copy(idx_hbm.at[pl.ds(base, _NPW)], idx_v)
    for j in range(_NCHUNK):
        pltpu.async_copy(
            table_hbm.at[idx_v.at[pl.ds(j * _CHUNK, _CHUNK)]],
            rows_v.at[pl.ds(j * _CHUNK, _CHUNK)],
            sem,
        ).wait()
    pltpu.sync_copy(rows_v, out_hbm.at[pl.ds(base, _NPW)])


_TBA = 2048
_TBB = 2048


def _gate_body(hs_ref, wg_ref, g_ref):
    hs = hs_ref[...].astype(jnp.bfloat16)
    g_ref[...] = jax.nn.sigmoid(
        lax.dot_general(hs, wg_ref[...].astype(jnp.bfloat16),
                        (((1,), (1,)), ((), ())),
                        preferred_element_type=jnp.float32))


def _fuse_body(e_ref, g_ref, wo_ref, nw_ref, out_ref):
    v = (e_ref[...] + g_ref[...]).astype(jnp.bfloat16)
    y = lax.dot_general(v, wo_ref[...].astype(jnp.bfloat16),
                        (((1,), (1,)), ((), ())),
                        preferred_element_type=jnp.float32)
    var = jnp.mean(y * y, axis=-1, keepdims=True)
    out_ref[...] = y * lax.rsqrt(var + 1e-6) * nw_ref[...]


def kernel(hidden_states, input_ids, memory, W_gate, W_out, norm_w):
    hs = hidden_states.reshape(N, HIDDEN)
    ids = input_ids.astype(jnp.int32).reshape(N)

    e = _sc_gather(memory, ids)

    g = pl.pallas_call(
        _gate_body,
        grid=(N // _TBA,),
        in_specs=[
            pl.BlockSpec((_TBA, HIDDEN), lambda i: (i, 0)),
            pl.BlockSpec((MEM, HIDDEN), lambda i: (0, 0)),
        ],
        out_specs=pl.BlockSpec((_TBA, MEM), lambda i: (i, 0)),
        out_shape=jax.ShapeDtypeStruct((N, MEM), jnp.float32),
    )(hs, W_gate)

    out = pl.pallas_call(
        _fuse_body,
        grid=(N // _TBB,),
        in_specs=[
            pl.BlockSpec((_TBB, MEM), lambda i: (i, 0)),
            pl.BlockSpec((_TBB, MEM), lambda i: (i, 0)),
            pl.BlockSpec((HIDDEN, MEM), lambda i: (0, 0)),
            pl.BlockSpec((1, HIDDEN), lambda i: (0, 0)),
        ],
        out_specs=pl.BlockSpec((_TBB, HIDDEN), lambda i: (i, 0)),
        out_shape=jax.ShapeDtypeStruct((N, HIDDEN), jnp.float32),
    )(e, g, W_out, norm_w.reshape(1, HIDDEN))

    return out.reshape(B, S, HIDDEN)


# speedup vs baseline: 1.1707x; 1.0072x over previous
import functools

import jax
import jax.numpy as jnp
from jax import lax
from jax.experimental import pallas as pl
from jax.experimental.pallas import tpu as pltpu
from jax.experimental.pallas import tpu_sc as plsc

VOCAB = 100000
HIDDEN = 2048
MEM = 128
B, S = 4, 4096
N = B * S


_info = plsc.get_sparse_core_info()
_NC, _NS = _info.num_cores, _info.num_subcores
_NW = _NC * _NS
_NPW = N // _NW
_CHUNK = 128
_NCHUNK = _NPW // _CHUNK


@functools.partial(
    pl.kernel,
    mesh=plsc.VectorSubcoreMesh(core_axis_name="c", subcore_axis_name="s"),
    out_type=jax.ShapeDtypeStruct((N, MEM), jnp.float32),
    scratch_types=[
        pltpu.VMEM((_NPW,), jnp.int32),
        pltpu.VMEM((_NPW, MEM), jnp.float32),
        pltpu.SemaphoreType.DMA,
    ],
)
def _sc_gather(table_hbm, idx_hbm, out_hbm, idx_v, rows_v, sem):
    wid = lax.axis_index("s") * _NC + lax.axis_index("c")
    base = wid * _NPW
    pltpu.sync_copy(idx_hbm.at[pl.ds(base, _NPW)], idx_v)
    for j in range(_NCHUNK):
        pltpu.async_copy(
            table_hbm.at[idx_v.at[pl.ds(j * _CHUNK, _CHUNK)]],
            rows_v.at[pl.ds(j * _CHUNK, _CHUNK)],
            sem,
        ).wait()
    pltpu.sync_copy(rows_v, out_hbm.at[pl.ds(base, _NPW)])


_TBA = 1024
_TBB = 1024


def _gate_body(hs_ref, wg_ref, g_ref):
    hs = hs_ref[...].astype(jnp.bfloat16)
    g_ref[...] = jax.nn.sigmoid(
        lax.dot_general(hs, wg_ref[...].astype(jnp.bfloat16),
                        (((1,), (1,)), ((), ())),
                        preferred_element_type=jnp.float32))


def _fuse_body(e_ref, g_ref, wo_ref, nw_ref, out_ref):
    v = (e_ref[...] + g_ref[...]).astype(jnp.bfloat16)
    y = lax.dot_general(v, wo_ref[...].astype(jnp.bfloat16),
                        (((1,), (1,)), ((), ())),
                        preferred_element_type=jnp.float32)
    var = jnp.mean(y * y, axis=-1, keepdims=True)
    out_ref[...] = y * lax.rsqrt(var + 1e-6) * nw_ref[...]


def kernel(hidden_states, input_ids, memory, W_gate, W_out, norm_w):
    hs = hidden_states.reshape(N, HIDDEN)
    ids = input_ids.astype(jnp.int32).reshape(N)

    e = _sc_gather(memory, ids)

    g = pl.pallas_call(
        _gate_body,
        grid=(N // _TBA,),
        in_specs=[
            pl.BlockSpec((_TBA, HIDDEN), lambda i: (i, 0)),
            pl.BlockSpec((MEM, HIDDEN), lambda i: (0, 0)),
        ],
        out_specs=pl.BlockSpec((_TBA, MEM), lambda i: (i, 0)),
        out_shape=jax.ShapeDtypeStruct((N, MEM), jnp.float32),
    )(hs, W_gate)

    out = pl.pallas_call(
        _fuse_body,
        grid=(N // _TBB,),
        in_specs=[
            pl.BlockSpec((_TBB, MEM), lambda i: (i, 0)),
            pl.BlockSpec((_TBB, MEM), lambda i: (i, 0)),
            pl.BlockSpec((HIDDEN, MEM), lambda i: (0, 0)),
            pl.BlockSpec((1, HIDDEN), lambda i: (0, 0)),
        ],
        out_specs=pl.BlockSpec((_TBB, HIDDEN), lambda i: (i, 0)),
        out_shape=jax.ShapeDtypeStruct((N, HIDDEN), jnp.float32),
    )(e, g, W_out, norm_w.reshape(1, HIDDEN))

    return out.reshape(B, S, HIDDEN)
